# Initial kernel scaffold; baseline (speedup 1.0000x reference)
import jax
import jax.numpy as jnp
from jax.experimental import pallas as pl


def kernel(x_review, x_user, x_product, ei_user_review, ei_product_review, ei_review_user, ei_review_product, params):
    raise NotImplementedError("write your pallas kernel here")


# speedup vs baseline: 1.2055x; 1.2055x over previous
import functools

import jax
import jax.numpy as jnp
from jax import lax
from jax.experimental import pallas as pl
from jax.experimental.pallas import tpu as pltpu
from jax.experimental.pallas import tpu_sc as plsc

F32 = jnp.float32
H = 128
N_REV = 50000
N_SMALL = 10000
E_PAD = 524288
IDXROWS = E_PAD // 128
NW = 32
ROWS_PER_W = IDXROWS // NW


def _sc_mesh():
    return plsc.VectorSubcoreMesh(core_axis_name="c", subcore_axis_name="s",
                                  num_cores=2, num_subcores=16)


def _zero_vmem(buf, nrows, ncols):
    npc = ncols // 16

    def body(i, carry):
        r = i // npc
        cs = (i % npc) * 16
        buf[r, pl.ds(cs, 16)] = jnp.zeros((16,), F32)
        return carry

    lax.fori_loop(0, nrows * npc, body, 0)


def _seg_small(table, sidx2d, didx2d, n_dst):
    acc_rows = n_dst + 112
    ch = 256
    chunks = ROWS_PER_W // 8
    zrpt = acc_rows // 16
    ob = 400
    nblk = n_dst // ob
    kmax = (nblk + 15) // 16

    @functools.partial(
        pl.kernel,
        out_type=jax.ShapeDtypeStruct((2, n_dst, H), F32),
        mesh=_sc_mesh(),
        scratch_types=[
            pltpu.VMEM_SHARED((acc_rows, H), F32),
            pltpu.VMEM((ch, H), F32),
            pltpu.VMEM((8, 128), jnp.int32),
            pltpu.VMEM((8, 128), jnp.int32),
            pltpu.SemaphoreType.DMA,
            pltpu.SemaphoreType.DMA,
            pltpu.SemaphoreType.DMA,
            pltpu.SemaphoreType.DMA,
        ],
    )
    def run(table_ref, s_ref, d_ref, out_ref, acc, rows, sidx, didx,
            sg0, sg1, ss0, ss1):
        c = lax.axis_index("c")
        s = lax.axis_index("s")
        w = s * 2 + c
        semg = (sg0, sg1)
        sems = (ss0, ss1)
        _zero_vmem(rows, ch, H)
        z0 = s * zrpt
        off = 0
        while off < zrpt:
            n = min(ch, zrpt - off)
            pltpu.sync_copy(rows.at[pl.ds(0, n)], acc.at[pl.ds(z0 + off, n)])
            off += n
        plsc.subcore_barrier()

        base = w * ROWS_PER_W

        def chunk(g, carry):
            rb = base + g * 8
            pltpu.sync_copy(s_ref.at[pl.ds(rb, 8)], sidx)
            pltpu.sync_copy(d_ref.at[pl.ds(rb, 8)], didx)
            dg = [None] * 8
            dsc = [None] * 8
            for j in range(8):
                b = j % 2
                if j >= 2:
                    dsc[j - 2].wait()
                dg[j] = pltpu.async_copy(
                    table_ref.at[sidx.at[j]],
                    rows.at[pl.ds(b * 128, 128)], semg[b])
                if j >= 1:
                    b2 = (j - 1) % 2
                    dg[j - 1].wait()
                    dsc[j - 1] = pltpu.async_copy(
                        rows.at[pl.ds(b2 * 128, 128)],
                        acc.at[didx.at[j - 1]], sems[b2], add=True)
            dg[7].wait()
            dsc[7] = pltpu.async_copy(rows.at[pl.ds(128, 128)],
                                      acc.at[didx.at[7]], sems[1], add=True)
            dsc[6].wait()
            dsc[7].wait()
            return carry

        lax.fori_loop(0, chunks, chunk, 0)
        plsc.subcore_barrier()
        for k in range(kmax):
            bid = s * kmax + k

            @pl.when(bid < nblk)
            def _copy():
                pltpu.sync_copy(acc.at[pl.ds(bid * ob, ob)],
                                out_ref.at[c, pl.ds(bid * ob, ob)])

    return run(table, sidx2d, didx2d)


def _seg_big(table4, sidx2d, didx2d, n_dst):
    acc_rows = n_dst + 48
    w32 = 32
    ch = 256
    chunks = ROWS_PER_W // 8
    zrpt = acc_rows // 16
    ob = 400
    nblk = n_dst // ob
    kmax = (nblk + 15) // 16

    @functools.partial(
        pl.kernel,
        out_type=jax.ShapeDtypeStruct((2, 4, n_dst, w32), F32),
        mesh=_sc_mesh(),
        scratch_types=[
            pltpu.VMEM_SHARED((acc_rows, w32), F32),
            pltpu.VMEM((ch, w32), F32),
            pltpu.VMEM((8, 128), jnp.int32),
            pltpu.VMEM((8, 128), jnp.int32),
            pltpu.VMEM((8, 128), jnp.int32),
            pltpu.SemaphoreType.DMA,
            pltpu.SemaphoreType.DMA,
            pltpu.SemaphoreType.DMA,
            pltpu.SemaphoreType.DMA,
        ],
        compiler_params=pltpu.CompilerParams(use_tc_tiling_on_sc=False),
    )
    def run(table_ref, s_ref, d_ref, out_ref, acc, rows, sidx, didx,
            gidx, sg0, sg1, ss0, ss1):
        c = lax.axis_index("c")
        s = lax.axis_index("s")
        w = s * 2 + c
        semg = (sg0, sg1)
        sems = (ss0, ss1)
        base = w * ROWS_PER_W

        for q in range(4):
            _zero_vmem(rows, ch, w32)
            z0 = s * zrpt
            off = 0
            while off < zrpt:
                n = min(ch, zrpt - off)
                pltpu.sync_copy(rows.at[pl.ds(0, n)],
                                acc.at[pl.ds(z0 + off, n)])
                off += n
            plsc.subcore_barrier()

            def chunk(g, carry):
                rb = base + g * 8
                pltpu.sync_copy(s_ref.at[pl.ds(rb, 8)], sidx)
                pltpu.sync_copy(d_ref.at[pl.ds(rb, 8)], didx)

                def mkidx(i, cc):
                    r = i // 8
                    cs = (i % 8) * 16
                    gidx[r, pl.ds(cs, 16)] = sidx[r, pl.ds(cs, 16)] * 4 + q
                    return cc

                lax.fori_loop(0, 64, mkidx, 0)
                dg = [None] * 8
                dsc = [None] * 8
                for sb in range(8):
                    b = sb % 2
                    if sb >= 2:
                        dsc[sb - 2].wait()
                    dg[sb] = pltpu.async_copy(
                        table_ref.at[gidx.at[sb]],
                        rows.at[pl.ds(b * 128, 128)], semg[b])
                    if sb >= 1:
                        b2 = (sb - 1) % 2
                        dg[sb - 1].wait()
                        dsc[sb - 1] = pltpu.async_copy(
                            rows.at[pl.ds(b2 * 128, 128)],
                            acc.at[didx.at[sb - 1]], sems[b2], add=True)
                dg[7].wait()
                dsc[7] = pltpu.async_copy(
                    rows.at[pl.ds(128, 128)],
                    acc.at[didx.at[7]], sems[1], add=True)
                dsc[6].wait()
                dsc[7].wait()
                return carry

            lax.fori_loop(0, chunks, chunk, 0)
            plsc.subcore_barrier()
            for k in range(kmax):
                bid = s * kmax + k

                @pl.when(bid < nblk)
                def _copy():
                    pltpu.sync_copy(acc.at[pl.ds(bid * ob, ob)],
                                    out_ref.at[c, q, pl.ds(bid * ob, ob)])

            plsc.subcore_barrier()

    return run(table4, sidx2d, didx2d)


def _counts(d_ur, d_pr, d_ru, d_rp):
    out_big = jax.ShapeDtypeStruct((2, N_REV, 16), F32)
    out_small = jax.ShapeDtypeStruct((2, N_SMALL, 16), F32)

    @functools.partial(
        pl.kernel,
        out_type=(out_big, out_big, out_small, out_small),
        mesh=_sc_mesh(),
        scratch_types=[
            pltpu.VMEM_SHARED((N_REV + 48, 16), F32),
            pltpu.VMEM_SHARED((N_SMALL + 112, 16), F32),
            pltpu.VMEM((128, 16), F32),
            pltpu.VMEM((1024, 16), F32),
            pltpu.VMEM((8, 128), jnp.int32),
            pltpu.SemaphoreType.DMA,
        ],
        compiler_params=pltpu.CompilerParams(use_tc_tiling_on_sc=False),
    )
    def run(ur_ref, pr_ref, ru_ref, rp_ref, o_ur, o_pr, o_ru, o_rp,
            acc_r, acc_u, ones, zbuf, didx, sem):
        c = lax.axis_index("c")
        s = lax.axis_index("s")
        w = s * 2 + c
        _zero_vmem(zbuf, 1024, 16)

        def fill_ones(i, carry):
            ones[i, pl.ds(0, 16)] = jnp.ones((16,), F32)
            return carry

        lax.fori_loop(0, 128, fill_ones, 0)
        base = w * ROWS_PER_W

        for d_ref, o_ref, acc, n_dst, pad in (
            (ur_ref, o_ur, acc_r, N_REV, 48),
            (pr_ref, o_pr, acc_r, N_REV, 48),
            (ru_ref, o_ru, acc_u, N_SMALL, 112),
            (rp_ref, o_rp, acc_u, N_SMALL, 112),
        ):
            zrpt = (n_dst + pad) // 16
            z0 = s * zrpt
            off = 0
            while off < zrpt:
                n = min(1024, zrpt - off)
                pltpu.sync_copy(zbuf.at[pl.ds(0, n)],
                                acc.at[pl.ds(z0 + off, n)])
                off += n
            plsc.subcore_barrier()

            def chunk(g, carry):
                rb = base + g * 8
                pltpu.sync_copy(d_ref.at[pl.ds(rb, 8)], didx)
                for j in range(8):
                    pltpu.sync_copy(ones, acc.at[didx.at[j]], add=True)
                return carry

            lax.fori_loop(0, ROWS_PER_W // 8, chunk, 0)
            plsc.subcore_barrier()
            nblk = n_dst // 400
            kmax = (nblk + 15) // 16
            for k in range(kmax):
                bid = s * kmax + k

                @pl.when(bid < nblk)
                def _copy():
                    pltpu.sync_copy(acc.at[pl.ds(bid * 400, 400)],
                                    o_ref.at[c, pl.ds(bid * 400, 400)])

            plsc.subcore_barrier()

    return run(d_ur, d_pr, d_ru, d_rp)


_BM = 2000


def _mm_relu(x, wt, b):
    m = x.shape[0]

    def body(x_ref, w_ref, b_ref, o_ref):
        o_ref[...] = jnp.maximum(
            jnp.dot(x_ref[...], w_ref[...], preferred_element_type=F32)
            + b_ref[...], 0.0)

    return pl.pallas_call(
        body,
        grid=(m // _BM,),
        in_specs=[
            pl.BlockSpec((_BM, H), lambda i: (i, 0)),
            pl.BlockSpec((H, H), lambda i: (0, 0)),
            pl.BlockSpec((1, H), lambda i: (0, 0)),
        ],
        out_specs=pl.BlockSpec((_BM, H), lambda i: (i, 0)),
        out_shape=jax.ShapeDtypeStruct((m, H), F32),
    )(x, wt, b)


def _sage1(s0, s1, c0, c1, h, wlt, bl, wrt):
    m = h.shape[0]

    def body(s0_ref, s1_ref, c0_ref, c1_ref, h_ref, wl_ref, bl_ref, wr_ref,
             o_ref):
        cnt = c0_ref[...][:, 0:1] + c1_ref[...][:, 0:1]
        mean = (s0_ref[...] + s1_ref[...]) / jnp.maximum(cnt, 1.0)
        o_ref[...] = (
            jnp.dot(mean, wl_ref[...], preferred_element_type=F32)
            + bl_ref[...]
            + jnp.dot(h_ref[...], wr_ref[...], preferred_element_type=F32))

    return pl.pallas_call(
        body,
        grid=(m // _BM,),
        in_specs=[
            pl.BlockSpec((_BM, H), lambda i: (i, 0)),
            pl.BlockSpec((_BM, H), lambda i: (i, 0)),
            pl.BlockSpec((_BM, 16), lambda i: (i, 0)),
            pl.BlockSpec((_BM, 16), lambda i: (i, 0)),
            pl.BlockSpec((_BM, H), lambda i: (i, 0)),
            pl.BlockSpec((H, H), lambda i: (0, 0)),
            pl.BlockSpec((1, H), lambda i: (0, 0)),
            pl.BlockSpec((H, H), lambda i: (0, 0)),
        ],
        out_specs=pl.BlockSpec((_BM, H), lambda i: (i, 0)),
        out_shape=jax.ShapeDtypeStruct((m, H), F32),
    )(s0, s1, c0, c1, h, wlt, bl, wrt)


def _sage2(ma0, ma1, ca0, ca1, mb0, mb1, cb0, cb1, h, wlat, wlbt, wrt, bias):
    m = h.shape[0]

    def body(a0, a1, ka0, ka1, b0, b1, kb0, kb1, h_ref, wla, wlb, wr, bz,
             o_ref):
        ca = ka0[...][:, 0:1] + ka1[...][:, 0:1]
        cb = kb0[...][:, 0:1] + kb1[...][:, 0:1]
        mean_a = (a0[...] + a1[...]) / jnp.maximum(ca, 1.0)
        mean_b = (b0[...] + b1[...]) / jnp.maximum(cb, 1.0)
        o_ref[...] = (
            jnp.dot(mean_a, wla[...], preferred_element_type=F32)
            + jnp.dot(mean_b, wlb[...], preferred_element_type=F32)
            + jnp.dot(h_ref[...], wr[...], preferred_element_type=F32)
            + bz[...])

    specs_feat = pl.BlockSpec((_BM, H), lambda i: (i, 0))
    specs_cnt = pl.BlockSpec((_BM, 16), lambda i: (i, 0))
    specs_w = pl.BlockSpec((H, H), lambda i: (0, 0))
    return pl.pallas_call(
        body,
        grid=(m // _BM,),
        in_specs=[
            specs_feat, specs_feat, specs_cnt, specs_cnt,
            specs_feat, specs_feat, specs_cnt, specs_cnt,
            specs_feat, specs_w, specs_w, specs_w,
            pl.BlockSpec((1, H), lambda i: (0, 0)),
        ],
        out_specs=specs_feat,
        out_shape=jax.ShapeDtypeStruct((m, H), F32),
    )(ma0, ma1, ca0, ca1, mb0, mb1, cb0, cb1, h, wlat, wlbt, wrt, bias)


def _bn_relu(x, w, b):
    m = x.shape[0]

    def stats(x_ref, o_ref):
        @pl.when(pl.program_id(0) == 0)
        def _init():
            o_ref[...] = jnp.zeros_like(o_ref)

        xv = x_ref[...]
        o_ref[0:1, :] += jnp.sum(xv, axis=0, keepdims=True)
        o_ref[1:2, :] += jnp.sum(xv * xv, axis=0, keepdims=True)

    st = pl.pallas_call(
        stats,
        grid=(m // _BM,),
        in_specs=[pl.BlockSpec((_BM, H), lambda i: (i, 0))],
        out_specs=pl.BlockSpec((8, H), lambda i: (0, 0)),
        out_shape=jax.ShapeDtypeStruct((8, H), F32),
    )(x)

    inv_m = 1.0 / m

    def apply(x_ref, st_ref, w_ref, b_ref, o_ref):
        mean = st_ref[0:1, :] * inv_m
        var = st_ref[1:2, :] * inv_m - mean * mean
        inv = lax.rsqrt(var + 1e-5)
        o_ref[...] = jnp.maximum(
            (x_ref[...] - mean) * inv * w_ref[...] + b_ref[...], 0.0)

    return pl.pallas_call(
        apply,
        grid=(m // _BM,),
        in_specs=[
            pl.BlockSpec((_BM, H), lambda i: (i, 0)),
            pl.BlockSpec((8, H), lambda i: (0, 0)),
            pl.BlockSpec((1, H), lambda i: (0, 0)),
            pl.BlockSpec((1, H), lambda i: (0, 0)),
        ],
        out_specs=pl.BlockSpec((_BM, H), lambda i: (i, 0)),
        out_shape=jax.ShapeDtypeStruct((m, H), F32),
    )(x, st, w, b)


def _classifier(x, w1t, b1, w2t, b2):
    m = x.shape[0]

    def body(x_ref, w1_ref, b1_ref, w2_ref, b2_ref, o_ref):
        z = jnp.maximum(
            jnp.dot(x_ref[...], w1_ref[...], preferred_element_type=F32)
            + b1_ref[...], 0.0)
        o_ref[...] = jnp.dot(z, w2_ref[...],
                             preferred_element_type=F32) + b2_ref[...]

    return pl.pallas_call(
        body,
        grid=(m // _BM,),
        in_specs=[
            pl.BlockSpec((_BM, H), lambda i: (i, 0)),
            pl.BlockSpec((H, 64), lambda i: (0, 0)),
            pl.BlockSpec((1, 64), lambda i: (0, 0)),
            pl.BlockSpec((64, 8), lambda i: (0, 0)),
            pl.BlockSpec((1, 8), lambda i: (0, 0)),
        ],
        out_specs=pl.BlockSpec((_BM, 8), lambda i: (i, 0)),
        out_shape=jax.ShapeDtypeStruct((m, 8), F32),
    )(x, w1t, b1, w2t, b2)


def _prep_edges(ei, dead):
    e = ei.shape[1]
    pad = E_PAD - e
    src = jnp.concatenate([ei[0], jnp.zeros((pad,), jnp.int32)])
    dst = jnp.concatenate([ei[1], jnp.full((pad,), dead, jnp.int32)])
    return src.reshape(IDXROWS, 128), dst.reshape(IDXROWS, 128)


def _big_to_mean(g):
    n = g.shape[2]
    return (g[0].transpose(1, 0, 2).reshape(n, H),
            g[1].transpose(1, 0, 2).reshape(n, H))


def kernel(x_review, x_user, x_product, ei_user_review, ei_product_review,
           ei_review_user, ei_review_product, params):
    p = params
    s_ur, d_ur = _prep_edges(ei_user_review, N_REV)
    s_pr, d_pr = _prep_edges(ei_product_review, N_REV)
    s_ru, d_ru = _prep_edges(ei_review_user, N_SMALL)
    s_rp, d_rp = _prep_edges(ei_review_product, N_SMALL)

    h_r = _mm_relu(x_review, p['proj']['review']['W'].T,
                   p['proj']['review']['b'][None, :])
    h_u = _mm_relu(x_user, p['proj']['user']['W'].T,
                   p['proj']['user']['b'][None, :])
    h_p = _mm_relu(x_product, p['proj']['product']['W'].T,
                   p['proj']['product']['b'][None, :])

    c_ur, c_pr, c_ru, c_rp = _counts(d_ur, d_pr, d_ru, d_rp)

    def conv_review(hu, hp, hr, cp):
        g_u = _seg_big(hu.reshape(-1, 32), s_ur, d_ur, N_REV)
        g_p = _seg_big(hp.reshape(-1, 32), s_pr, d_pr, N_REV)
        ma0, ma1 = _big_to_mean(g_u)
        mb0, mb1 = _big_to_mean(g_p)
        a, b = cp['user_review'], cp['product_review']
        return _sage2(
            ma0, ma1, c_ur[0], c_ur[1], mb0, mb1, c_pr[0], c_pr[1], hr,
            0.5 * a['lin_l']['W'].T, 0.5 * b['lin_l']['W'].T,
            0.5 * (a['lin_r']['W'] + b['lin_r']['W']).T,
            (0.5 * (a['lin_l']['b'] + b['lin_l']['b']))[None, :])

    r1 = conv_review(h_u, h_p, h_r, p['conv1'])
    g_ru = _seg_small(h_r, s_ru, d_ru, N_SMALL)
    g_rp = _seg_small(h_r, s_rp, d_rp, N_SMALL)
    cu = p['conv1']['review_user']
    u1 = _sage1(g_ru[0], g_ru[1], c_ru[0], c_ru[1], h_u,
                cu['lin_l']['W'].T, cu['lin_l']['b'][None, :],
                cu['lin_r']['W'].T)
    cv = p['conv1']['review_product']
    p1 = _sage1(g_rp[0], g_rp[1], c_rp[0], c_rp[1], h_p,
                cv['lin_l']['W'].T, cv['lin_l']['b'][None, :],
                cv['lin_r']['W'].T)
    r1 = _bn_relu(r1, p['bn1']['review']['w'][None, :],
                  p['bn1']['review']['b'][None, :])
    u1 = _bn_relu(u1, p['bn1']['user']['w'][None, :],
                  p['bn1']['user']['b'][None, :])
    p1 = _bn_relu(p1, p['bn1']['product']['w'][None, :],
                  p['bn1']['product']['b'][None, :])

    r2 = conv_review(u1, p1, r1, p['conv2'])
    r2 = _bn_relu(r2, p['bn2']['review']['w'][None, :],
                  p['bn2']['review']['b'][None, :])

    w2 = jnp.zeros((8, 64), F32).at[:2].set(p['cls']['l2']['W'])
    b2 = jnp.zeros((8,), F32).at[:2].set(p['cls']['l2']['b'])
    out = _classifier(r2, p['cls']['l1']['W'].T,
                      p['cls']['l1']['b'][None, :], w2.T, b2[None, :])
    return out[:, :2]
